# Initial kernel scaffold; baseline (speedup 1.0000x reference)
import jax
import jax.numpy as jnp
from jax.experimental import pallas as pl


def kernel(x, rank_masks, edge_index, he_idxs):
    raise NotImplementedError("write your pallas kernel here")


# speedup vs baseline: 3.1003x; 3.1003x over previous
import jax
import jax.numpy as jnp
from jax import lax
from jax.experimental import pallas as pl
from jax.experimental.pallas import tpu as pltpu
from jax.experimental.pallas import tpu_sc as plsc

N, H, E, D, R = 10000, 10000, 160000, 128, 8
NC, NS, NW = 2, 16, 32
CH = 128
NCHUNK = E // CH
QC = 80
NQ = H // QC
RM1 = R - 1

_mesh = plsc.VectorSubcoreMesh(core_axis_name="c", subcore_axis_name="s")
f32 = jnp.float32
i32 = jnp.int32


def _fill_zeros(zbuf):
    def body(i, carry):
        for j in range(D // 16):
            zbuf[i, pl.ds(j * 16, 16)] = jnp.zeros((16,), f32)
        return carry
    lax.fori_loop(0, QC, body, 0)


def _zero_shared(s, zbuf, sh):
    def body(t, carry):
        q = s + NS * t

        @pl.when(q < NQ)
        def _():
            pltpu.sync_copy(zbuf, sh.at[pl.ds(q * QC, QC)])
        return carry
    lax.fori_loop(0, NQ // NS + 1, body, 0)


def _hf_body(x_hbm, nidx_hbm, hidx_hbm, hfp_hbm, nbuf, hbuf, rows, zbuf, sem,
             hf_sh):
    c = lax.axis_index("c")
    s = lax.axis_index("s")
    wid = c * NS + s
    _fill_zeros(zbuf)
    _zero_shared(s, zbuf, hf_sh)
    plsc.subcore_barrier()

    def body(k, carry):
        ch = wid + NW * k

        @pl.when(ch < NCHUNK)
        def _():
            pltpu.sync_copy(nidx_hbm.at[pl.ds(ch * CH, CH)], nbuf)
            pltpu.sync_copy(hidx_hbm.at[pl.ds(ch * CH, CH)], hbuf)
            pltpu.async_copy(x_hbm.at[nbuf], rows, sem).wait()
            pltpu.sync_copy(rows, hf_sh.at[hbuf], add=True)
        return carry

    lax.fori_loop(0, NCHUNK // NW + 1, body, 0)
    plsc.subcore_barrier()

    def wout(t, carry):
        q = s + NS * t

        @pl.when(q < NQ)
        def _():
            @pl.when(c == 0)
            def _():
                pltpu.sync_copy(hf_sh.at[pl.ds(q * QC, QC)],
                                hfp_hbm.at[0, pl.ds(q * QC, QC)])

            @pl.when(c == 1)
            def _():
                pltpu.sync_copy(hf_sh.at[pl.ds(q * QC, QC)],
                                hfp_hbm.at[1, pl.ds(q * QC, QC)])
        return carry

    lax.fori_loop(0, NQ // NS + 1, wout, 0)


def _flat_body(hfp_hbm, rmb_hbm, flat_hbm, a, b, w, fl):
    c = lax.axis_index("c")
    s = lax.axis_index("s")
    wid = c * NS + s
    nch = H // 8

    def body(k, carry):
        ch = wid + NW * k

        @pl.when(ch < nch)
        def _():
            r0 = ch * 8
            pltpu.sync_copy(hfp_hbm.at[0, pl.ds(r0, 8)], a)
            pltpu.sync_copy(hfp_hbm.at[1, pl.ds(r0, 8)], b)
            pltpu.sync_copy(rmb_hbm.at[pl.ds(r0, 8)], w)
            for i in range(8):
                for j in range(D // 16):
                    hv = a[i, pl.ds(j * 16, 16)] + b[i, pl.ds(j * 16, 16)]
                    for r in range(RM1):
                        fl[i * RM1 + r, pl.ds(j * 16, 16)] = (
                            hv * w[i, pl.ds(r * 16, 16)])
            pltpu.sync_copy(fl, flat_hbm.at[pl.ds(ch * 8 * RM1, 8 * RM1)])
        return carry

    lax.fori_loop(0, nch // NW + 1, body, 0)


def _out_body(flat_hbm, nidx_hbm, hidx_hbm, x_hbm, out_hbm, nbuf, hbuf, gbuf,
              rows, zbuf, sem, acc_sh):
    c = lax.axis_index("c")
    s = lax.axis_index("s")
    _fill_zeros(zbuf)

    for r in range(RM1):
        core_r = 0 if r < 4 else 1

        @pl.when(c == core_r)
        def _(r=r):
            _zero_shared(s, zbuf, acc_sh)
            plsc.subcore_barrier()

            def body(k, carry):
                ch = s + NS * k

                @pl.when(ch < NCHUNK)
                def _():
                    pltpu.sync_copy(nidx_hbm.at[pl.ds(ch * CH, CH)], nbuf)
                    pltpu.sync_copy(hidx_hbm.at[pl.ds(ch * CH, CH)], hbuf)
                    for j in range(CH // 16):
                        gbuf[pl.ds(j * 16, 16)] = (
                            hbuf[pl.ds(j * 16, 16)] * RM1 + r)
                    pltpu.async_copy(flat_hbm.at[gbuf], rows, sem).wait()
                    pltpu.sync_copy(rows, acc_sh.at[nbuf], add=True)
                return carry

            lax.fori_loop(0, NCHUNK // NS + 1, body, 0)
            plsc.subcore_barrier()

            def wout(t, carry):
                q = s + NS * t

                @pl.when(q < NQ)
                def _():
                    pltpu.sync_copy(acc_sh.at[pl.ds(q * QC, QC)],
                                    out_hbm.at[r, pl.ds(q * QC, QC)])
                return carry

            lax.fori_loop(0, NQ // NS + 1, wout, 0)

    @pl.when(c == 1)
    def _():
        def xcopy(t, carry):
            q = s + NS * t

            @pl.when(q < NQ)
            def _():
                pltpu.sync_copy(x_hbm.at[pl.ds(q * QC, QC)],
                                rows.at[pl.ds(0, QC)])
                pltpu.sync_copy(rows.at[pl.ds(0, QC)],
                                out_hbm.at[RM1, pl.ds(q * QC, QC)])
            return carry

        lax.fori_loop(0, NQ // NS + 1, xcopy, 0)


_hf_call = pl.kernel(
    _hf_body,
    out_type=jax.ShapeDtypeStruct((2, H, D), f32),
    mesh=_mesh,
    scratch_types=[
        pltpu.VMEM((CH,), i32),
        pltpu.VMEM((CH,), i32),
        pltpu.VMEM((CH, D), f32),
        pltpu.VMEM((QC, D), f32),
        pltpu.SemaphoreType.DMA,
        pltpu.VMEM_SHARED((H, D), f32),
    ],
)

_flat_call = pl.kernel(
    _flat_body,
    out_type=jax.ShapeDtypeStruct((H * RM1, D), f32),
    mesh=_mesh,
    scratch_types=[
        pltpu.VMEM((8, D), f32),
        pltpu.VMEM((8, D), f32),
        pltpu.VMEM((8, RM1 * 16), f32),
        pltpu.VMEM((8 * RM1, D), f32),
    ],
)

_out_call = pl.kernel(
    _out_body,
    out_type=jax.ShapeDtypeStruct((R, N, D), f32),
    mesh=_mesh,
    scratch_types=[
        pltpu.VMEM((CH,), i32),
        pltpu.VMEM((CH,), i32),
        pltpu.VMEM((CH,), i32),
        pltpu.VMEM((CH, D), f32),
        pltpu.VMEM((QC, D), f32),
        pltpu.SemaphoreType.DMA,
        pltpu.VMEM_SHARED((N, D), f32),
    ],
)


@jax.jit
def kernel(x, rank_masks, edge_index, he_idxs):
    rm = rank_masks[:, he_idxs]
    rmb = jnp.broadcast_to(rm[:RM1].T[:, :, None], (H, RM1, 16))
    rmb = rmb.reshape(H, RM1 * 16).astype(f32)
    nidx = edge_index[0]
    hidx = edge_index[1]
    hfp = _hf_call(x, nidx, hidx)
    flat = _flat_call(hfp, rmb)
    out = _out_call(flat, nidx, hidx, x)
    return out.transpose(1, 0, 2)


# speedup vs baseline: 5.8767x; 1.8955x over previous
import jax
import jax.numpy as jnp
from jax import lax
from jax.experimental import pallas as pl
from jax.experimental.pallas import tpu as pltpu
from jax.experimental.pallas import tpu_sc as plsc

N, H, E, D, R = 10000, 10000, 160000, 128, 8
NC, NS, NW = 2, 16, 32
CH = 128
NCHUNK = E // CH
QC = 80
NQ = H // QC
RM1 = R - 1

_mesh = plsc.VectorSubcoreMesh(core_axis_name="c", subcore_axis_name="s")
f32 = jnp.float32
i32 = jnp.int32


def _fill_zeros(zbuf):
    def body(i, carry):
        for j in range(D // 16):
            zbuf[i, pl.ds(j * 16, 16)] = jnp.zeros((16,), f32)
        return carry
    lax.fori_loop(0, QC, body, 0)


def _zero_shared(s, zbuf, sh, sem):
    for t in range(NQ // NS + 1):
        q = s + NS * t

        @pl.when(q < NQ)
        def _(q=q):
            pltpu.async_copy(zbuf, sh.at[pl.ds(q * QC, QC)], sem)
    for t in range(NQ // NS + 1):
        q = s + NS * t

        @pl.when(q < NQ)
        def _(q=q):
            pltpu.make_async_copy(zbuf, sh.at[pl.ds(q * QC, QC)], sem).wait()


def _edge_pipeline(s_first, stride, nt, eidx_hbm, ebuf, isem,
                   gidx_fn, g_start, g_wait, s_start, s_wait):
    def valid(t):
        return s_first + stride * t < NCHUNK

    def idx_start(t, e4):
        @pl.when(valid(t))
        def _():
            ch = s_first + stride * t
            pltpu.async_copy(eidx_hbm.at[:, pl.ds(ch * CH, CH)], ebuf[e4],
                             isem[e4])

    def idx_wait(t, e4):
        @pl.when(valid(t))
        def _():
            ch = s_first + stride * t
            pltpu.make_async_copy(eidx_hbm.at[:, pl.ds(ch * CH, CH)],
                                  ebuf[e4], isem[e4]).wait()

    idx_start(0, 0)
    idx_start(1, 1)
    idx_wait(0, 0)

    @pl.when(valid(0))
    def _():
        gidx_fn(0, 0)
        g_start(0, 0)

    def iter_t(t, off):
        b, e4 = off % 2, off % 4
        bn, e4n1, e4n2 = 1 - b, (off + 1) % 4, (off + 2) % 4

        @pl.when(valid(t))
        def _():
            g_wait(b, e4)
            s_start(b, e4)
            idx_start(t + 2, e4n2)

        idx_wait(t + 1, e4n1)

        @pl.when(valid(t + 1))
        def _():
            gidx_fn(bn, e4n1)

        @pl.when((t >= 1) & valid(t - 1))
        def _():
            s_wait(bn, e4n1)

        @pl.when(valid(t + 1))
        def _():
            g_start(bn, e4n1)

    def body(k4, carry):
        t0 = 4 * k4
        for off in range(4):
            iter_t(t0 + off, off)
        return carry

    lax.fori_loop(0, (nt + 5) // 4, body, 0)


def _hf_body(x_hbm, eidx_hbm, hfp_hbm, e0, e1, e2, e3, rows0, rows1, zbuf,
             isem, gsem, ssem, zsem, hf_sh):
    c = lax.axis_index("c")
    s = lax.axis_index("s")
    wid = c * NS + s
    ebuf = [e0, e1, e2, e3]
    rows = [rows0, rows1]
    _fill_zeros(zbuf)
    _zero_shared(s, zbuf, hf_sh, zsem)
    plsc.subcore_barrier()

    def g_start(b, e4):
        pltpu.async_copy(x_hbm.at[ebuf[e4].at[0]], rows[b], gsem[b])

    def g_wait(b, e4):
        pltpu.make_async_copy(x_hbm.at[ebuf[e4].at[0]], rows[b],
                              gsem[b]).wait()

    def s_start(b, e4):
        pltpu.async_copy(rows[b], hf_sh.at[ebuf[e4].at[1]], ssem[b],
                         add=True)

    def s_wait(b, e4):
        pltpu.make_async_copy(rows[b], hf_sh.at[ebuf[e4].at[1]],
                              ssem[b]).wait()

    _edge_pipeline(wid, NW, NCHUNK // NW + 1, eidx_hbm, ebuf, isem,
                   lambda b, e4: None, g_start, g_wait, s_start, s_wait)
    plsc.subcore_barrier()

    def wo(fire):
        for t in range(NQ // NS + 1):
            q = s + NS * t

            @pl.when(q < NQ)
            def _(q=q):
                for cc in range(2):
                    @pl.when(c == cc)
                    def _(cc=cc, q=q):
                        cp = (pltpu.async_copy if fire
                              else lambda a, b2, sm:
                              pltpu.make_async_copy(a, b2, sm).wait())
                        cp(hf_sh.at[pl.ds(q * QC, QC)],
                           hfp_hbm.at[cc, pl.ds(q * QC, QC)], zsem)

    wo(True)
    wo(False)


def _flat_body(hfp_hbm, rmb_hbm, flat_hbm, ab0, ab1, w0, w1, fl0, fl1,
               isem, osem):
    c = lax.axis_index("c")
    s = lax.axis_index("s")
    wid = c * NS + s
    nch = H // 8
    ab = [ab0, ab1]
    w = [w0, w1]
    fl = [fl0, fl1]
    nt = nch // NW + 1

    def valid(t):
        return wid + NW * t < nch

    def in_start(t, b):
        @pl.when(valid(t))
        def _():
            r0 = (wid + NW * t) * 8
            pltpu.async_copy(hfp_hbm.at[:, pl.ds(r0, 8)], ab[b], isem[b])
            pltpu.async_copy(rmb_hbm.at[pl.ds(r0, 8)], w[b], isem[b])

    def in_wait(t, b):
        @pl.when(valid(t))
        def _():
            r0 = (wid + NW * t) * 8
            pltpu.make_async_copy(hfp_hbm.at[:, pl.ds(r0, 8)], ab[b],
                                  isem[b]).wait()
            pltpu.make_async_copy(rmb_hbm.at[pl.ds(r0, 8)], w[b],
                                  isem[b]).wait()

    def out_wait(t, b):
        @pl.when((t >= 0) & valid(t))
        def _():
            f0 = (wid + NW * t) * 8 * RM1
            pltpu.make_async_copy(fl[b], flat_hbm.at[pl.ds(f0, 8 * RM1)],
                                  osem[b]).wait()

    def iter_t(t, b):
        in_start(t + 1, 1 - b)
        in_wait(t, b)
        out_wait(t - 2, b)

        @pl.when(valid(t))
        def _():
            for i in range(8):
                for j in range(D // 16):
                    hv = (ab[b][0, i, pl.ds(j * 16, 16)]
                          + ab[b][1, i, pl.ds(j * 16, 16)])
                    for r in range(RM1):
                        fl[b][i * RM1 + r, pl.ds(j * 16, 16)] = (
                            hv * w[b][i, pl.ds(r * 16, 16)])
            f0 = (wid + NW * t) * 8 * RM1
            pltpu.async_copy(fl[b], flat_hbm.at[pl.ds(f0, 8 * RM1)], osem[b])

    in_start(0, 0)

    def body(k2, carry):
        t0 = 2 * k2
        iter_t(t0, 0)
        iter_t(t0 + 1, 1)
        return carry

    lax.fori_loop(0, nt // 2, body, 0)
    out_wait(nt - 2, 0)
    out_wait(nt - 1, 1)


def _out_body(flat_hbm, eidx_hbm, x_hbm, out_hbm, e0, e1, e2, e3, g0, g1,
              rows0, rows1, zbuf, isem, gsem, ssem, zsem, acc_sh):
    c = lax.axis_index("c")
    s = lax.axis_index("s")
    ebuf = [e0, e1, e2, e3]
    gbuf = [g0, g1]
    rows = [rows0, rows1]
    _fill_zeros(zbuf)

    def rank_pass(r, carry):
        _zero_shared(s, zbuf, acc_sh, zsem)
        plsc.subcore_barrier()

        def gidx_fn(b, e4):
            for j in range(CH // 16):
                gbuf[b][pl.ds(j * 16, 16)] = (
                    ebuf[e4][1, pl.ds(j * 16, 16)] * RM1 + r)

        def g_start(b, e4):
            pltpu.async_copy(flat_hbm.at[gbuf[b]], rows[b], gsem[b])

        def g_wait(b, e4):
            pltpu.make_async_copy(flat_hbm.at[gbuf[b]], rows[b],
                                  gsem[b]).wait()

        def s_start(b, e4):
            pltpu.async_copy(rows[b], acc_sh.at[ebuf[e4].at[0]], ssem[b],
                             add=True)

        def s_wait(b, e4):
            pltpu.make_async_copy(rows[b], acc_sh.at[ebuf[e4].at[0]],
                                  ssem[b]).wait()

        _edge_pipeline(s, NS, NCHUNK // NS + 1, eidx_hbm, ebuf, isem,
                       gidx_fn, g_start, g_wait, s_start, s_wait)
        plsc.subcore_barrier()

        for t in range(NQ // NS + 1):
            q = s + NS * t

            @pl.when(q < NQ)
            def _(q=q):
                pltpu.async_copy(acc_sh.at[pl.ds(q * QC, QC)],
                                 out_hbm.at[r, pl.ds(q * QC, QC)], zsem)
        for t in range(NQ // NS + 1):
            q = s + NS * t

            @pl.when(q < NQ)
            def _(q=q):
                pltpu.make_async_copy(acc_sh.at[pl.ds(q * QC, QC)],
                                      out_hbm.at[r, pl.ds(q * QC, QC)],
                                      zsem).wait()
        return carry

    lax.fori_loop(c * 4, 4 + c * 3, rank_pass, 0)

    @pl.when(c == 1)
    def _():
        for t in range(NQ // NS + 1):
            q = s + NS * t

            @pl.when(q < NQ)
            def _(q=q):
                pltpu.sync_copy(x_hbm.at[pl.ds(q * QC, QC)],
                                rows0.at[pl.ds(0, QC)])
                pltpu.sync_copy(rows0.at[pl.ds(0, QC)],
                                out_hbm.at[RM1, pl.ds(q * QC, QC)])


_hf_call = pl.kernel(
    _hf_body,
    out_type=jax.ShapeDtypeStruct((2, H, D), f32),
    mesh=_mesh,
    scratch_types=[
        pltpu.VMEM((2, CH), i32),
        pltpu.VMEM((2, CH), i32),
        pltpu.VMEM((2, CH), i32),
        pltpu.VMEM((2, CH), i32),
        pltpu.VMEM((CH, D), f32),
        pltpu.VMEM((CH, D), f32),
        pltpu.VMEM((QC, D), f32),
        [pltpu.SemaphoreType.DMA] * 4,
        [pltpu.SemaphoreType.DMA] * 2,
        [pltpu.SemaphoreType.DMA] * 2,
        pltpu.SemaphoreType.DMA,
        pltpu.VMEM_SHARED((H, D), f32),
    ],
)

_flat_call = pl.kernel(
    _flat_body,
    out_type=jax.ShapeDtypeStruct((H * RM1, D), f32),
    mesh=_mesh,
    scratch_types=[
        pltpu.VMEM((2, 8, D), f32),
        pltpu.VMEM((2, 8, D), f32),
        pltpu.VMEM((8, RM1 * 16), f32),
        pltpu.VMEM((8, RM1 * 16), f32),
        pltpu.VMEM((8 * RM1, D), f32),
        pltpu.VMEM((8 * RM1, D), f32),
        [pltpu.SemaphoreType.DMA] * 2,
        [pltpu.SemaphoreType.DMA] * 2,
    ],
)

_out_call = pl.kernel(
    _out_body,
    out_type=jax.ShapeDtypeStruct((R, N, D), f32),
    mesh=_mesh,
    scratch_types=[
        pltpu.VMEM((2, CH), i32),
        pltpu.VMEM((2, CH), i32),
        pltpu.VMEM((2, CH), i32),
        pltpu.VMEM((2, CH), i32),
        pltpu.VMEM((CH,), i32),
        pltpu.VMEM((CH,), i32),
        pltpu.VMEM((CH, D), f32),
        pltpu.VMEM((CH, D), f32),
        pltpu.VMEM((QC, D), f32),
        [pltpu.SemaphoreType.DMA] * 4,
        [pltpu.SemaphoreType.DMA] * 2,
        [pltpu.SemaphoreType.DMA] * 2,
        pltpu.SemaphoreType.DMA,
        pltpu.VMEM_SHARED((N, D), f32),
    ],
)


@jax.jit
def kernel(x, rank_masks, edge_index, he_idxs):
    rm = rank_masks[:, he_idxs]
    rmb = jnp.broadcast_to(rm[:RM1].T[:, :, None], (H, RM1, 16))
    rmb = rmb.reshape(H, RM1 * 16).astype(f32)
    hfp = _hf_call(x, edge_index)
    flat = _flat_call(hfp, rmb)
    out = _out_call(flat, edge_index, x)
    return out.transpose(1, 0, 2)


# speedup vs baseline: 6.2270x; 1.0596x over previous
import jax
import jax.numpy as jnp
from jax import lax
from jax.experimental import pallas as pl
from jax.experimental.pallas import tpu as pltpu
from jax.experimental.pallas import tpu_sc as plsc

N, H, E, D, R = 10000, 10000, 160000, 128, 8
NC, NS, NW = 2, 16, 32
CH = 128
NCHUNK = E // CH
QC = 80
NQ = H // QC
ZC = 80
NZ = H // ZC
RM1 = R - 1

_mesh = plsc.VectorSubcoreMesh(core_axis_name="c", subcore_axis_name="s")
f32 = jnp.float32
i32 = jnp.int32


def _fill_zeros(zbuf):
    def body(i, carry):
        for j in range(D // 16):
            zbuf[i, pl.ds(j * 16, 16)] = jnp.zeros((16,), f32)
        return carry
    lax.fori_loop(0, ZC, body, 0)


def _zero_shared(s, zbuf, sh, sem):
    for t in range(NZ // NS + 1):
        q = s + NS * t

        @pl.when(q < NZ)
        def _(q=q):
            pltpu.async_copy(zbuf, sh.at[pl.ds(q * ZC, ZC)], sem)
    for t in range(NZ // NS + 1):
        q = s + NS * t

        @pl.when(q < NZ)
        def _(q=q):
            pltpu.make_async_copy(zbuf, sh.at[pl.ds(q * ZC, ZC)], sem).wait()


def _edge_pipeline(s_first, stride, nt, eidx_hbm, ebuf, isem,
                   gidx_fn, g_start, g_wait, s_start, s_wait):
    def valid(t):
        return s_first + stride * t < NCHUNK

    def idx_start(t, e4):
        @pl.when(valid(t))
        def _():
            ch = s_first + stride * t
            pltpu.async_copy(eidx_hbm.at[:, pl.ds(ch * CH, CH)], ebuf[e4],
                             isem[e4])

    def idx_wait(t, e4):
        @pl.when(valid(t))
        def _():
            ch = s_first + stride * t
            pltpu.make_async_copy(eidx_hbm.at[:, pl.ds(ch * CH, CH)],
                                  ebuf[e4], isem[e4]).wait()

    idx_start(0, 0)
    idx_start(1, 1)
    idx_wait(0, 0)

    @pl.when(valid(0))
    def _():
        gidx_fn(0, 0)
        g_start(0, 0)

    def iter_t(t, off):
        b, e4 = off % 2, off % 4
        bn, e4n1, e4n2 = 1 - b, (off + 1) % 4, (off + 2) % 4

        @pl.when(valid(t))
        def _():
            g_wait(b, e4)
            s_start(b, e4)
            idx_start(t + 2, e4n2)

        idx_wait(t + 1, e4n1)

        @pl.when(valid(t + 1))
        def _():
            gidx_fn(bn, e4n1)

        @pl.when((t >= 1) & valid(t - 1))
        def _():
            s_wait(bn, e4n1)

        @pl.when(valid(t + 1))
        def _():
            g_start(bn, e4n1)

    def body(k4, carry):
        t0 = 4 * k4
        for off in range(4):
            iter_t(t0 + off, off)
        return carry

    lax.fori_loop(0, (nt + 5) // 4, body, 0)


def _hf_body(x_hbm, eidx_hbm, hfp_hbm, e0, e1, e2, e3, rows0, rows1,
             zbuf, isem, gsem, ssem, zsem, hf_sh):
    c = lax.axis_index("c")
    s = lax.axis_index("s")
    wid = c * NS + s
    ebuf = [e0, e1, e2, e3]
    rows = [rows0, rows1]
    _fill_zeros(zbuf)
    _zero_shared(s, zbuf, hf_sh, zsem)
    plsc.subcore_barrier()

    def g_start(b, e4):
        pltpu.async_copy(x_hbm.at[ebuf[e4].at[0]], rows[b], gsem[b])

    def g_wait(b, e4):
        pltpu.make_async_copy(x_hbm.at[ebuf[e4].at[0]], rows[b],
                              gsem[b]).wait()

    def s_start(b, e4):
        pltpu.async_copy(rows[b], hf_sh.at[ebuf[e4].at[1]], ssem[b],
                         add=True)

    def s_wait(b, e4):
        pltpu.make_async_copy(rows[b], hf_sh.at[ebuf[e4].at[1]],
                              ssem[b]).wait()

    _edge_pipeline(wid, NW, NCHUNK // NW + 1, eidx_hbm, ebuf, isem,
                   lambda b, e4: None, g_start, g_wait, s_start, s_wait)
    plsc.subcore_barrier()

    def wo(fire):
        for t in range(NQ // NS + 1):
            q = s + NS * t

            @pl.when(q < NQ)
            def _(q=q):
                for cc in range(2):
                    @pl.when(c == cc)
                    def _(cc=cc, q=q):
                        cp = (pltpu.async_copy if fire
                              else lambda a, b2, sm:
                              pltpu.make_async_copy(a, b2, sm).wait())
                        cp(hf_sh.at[pl.ds(q * QC, QC)],
                           hfp_hbm.at[cc, pl.ds(q * QC, QC)], zsem)

    wo(True)
    wo(False)


def _flat_body(hfp_hbm, rmb_hbm, flat_hbm, ab0, ab1, w0, w1, fl0, fl1,
               isem, osem):
    c = lax.axis_index("c")
    s = lax.axis_index("s")
    wid = c * NS + s
    nch = H // 8
    ab = [ab0, ab1]
    w = [w0, w1]
    fl = [fl0, fl1]
    nt = nch // NW + 1

    def valid(t):
        return wid + NW * t < nch

    def in_start(t, b):
        @pl.when(valid(t))
        def _():
            r0 = (wid + NW * t) * 8
            pltpu.async_copy(hfp_hbm.at[:, pl.ds(r0, 8)], ab[b], isem[b])
            pltpu.async_copy(rmb_hbm.at[pl.ds(r0, 8)], w[b], isem[b])

    def in_wait(t, b):
        @pl.when(valid(t))
        def _():
            r0 = (wid + NW * t) * 8
            pltpu.make_async_copy(hfp_hbm.at[:, pl.ds(r0, 8)], ab[b],
                                  isem[b]).wait()
            pltpu.make_async_copy(rmb_hbm.at[pl.ds(r0, 8)], w[b],
                                  isem[b]).wait()

    def out_wait(t, b):
        @pl.when((t >= 0) & valid(t))
        def _():
            f0 = (wid + NW * t) * 8 * RM1
            pltpu.make_async_copy(fl[b], flat_hbm.at[pl.ds(f0, 8 * RM1)],
                                  osem[b]).wait()

    def iter_t(t, b):
        in_start(t + 1, 1 - b)
        in_wait(t, b)
        out_wait(t - 2, b)

        @pl.when(valid(t))
        def _():
            for i in range(8):
                for j in range(D // 16):
                    hv = (ab[b][0, i, pl.ds(j * 16, 16)]
                          + ab[b][1, i, pl.ds(j * 16, 16)])
                    for r in range(RM1):
                        fl[b][i * RM1 + r, pl.ds(j * 16, 16)] = (
                            hv * w[b][i, pl.ds(r * 16, 16)])
            f0 = (wid + NW * t) * 8 * RM1
            pltpu.async_copy(fl[b], flat_hbm.at[pl.ds(f0, 8 * RM1)], osem[b])

    in_start(0, 0)

    def body(k2, carry):
        t0 = 2 * k2
        iter_t(t0, 0)
        iter_t(t0 + 1, 1)
        return carry

    lax.fori_loop(0, nt // 2, body, 0)
    out_wait(nt - 2, 0)
    out_wait(nt - 1, 1)


def _out_body(flat_hbm, eidx_hbm, x_hbm, outv_hbm, e0, e1, e2, e3, g0, g1,
              rows0, rows1, zbuf, ibase, ibuf, isem, gsem, ssem, zsem,
              acc_sh):
    c = lax.axis_index("c")
    s = lax.axis_index("s")
    ebuf = [e0, e1, e2, e3]
    gbuf = [g0, g1]
    rows = [rows0, rows1]
    _fill_zeros(zbuf)
    for j in range(QC // 16):
        ibase[pl.ds(j * 16, 16)] = (lax.iota(i32, 16) + 16 * j) * R

    def rank_pass(r, carry):
        _zero_shared(s, zbuf, acc_sh, zsem)
        plsc.subcore_barrier()

        def gidx_fn(b, e4):
            for j in range(CH // 16):
                gbuf[b][pl.ds(j * 16, 16)] = (
                    ebuf[e4][1, pl.ds(j * 16, 16)] * RM1 + r)

        def g_start(b, e4):
            pltpu.async_copy(flat_hbm.at[gbuf[b]], rows[b], gsem[b])

        def g_wait(b, e4):
            pltpu.make_async_copy(flat_hbm.at[gbuf[b]], rows[b],
                                  gsem[b]).wait()

        def s_start(b, e4):
            pltpu.async_copy(rows[b], acc_sh.at[ebuf[e4].at[0]], ssem[b],
                             add=True)

        def s_wait(b, e4):
            pltpu.make_async_copy(rows[b], acc_sh.at[ebuf[e4].at[0]],
                                  ssem[b]).wait()

        _edge_pipeline(s, NS, NCHUNK // NS + 1, eidx_hbm, ebuf, isem,
                       gidx_fn, g_start, g_wait, s_start, s_wait)
        plsc.subcore_barrier()

        for t in range(NQ // NS + 3):
            b = t % 2
            if t >= 2:
                qp = s + NS * (t - 2)

                @pl.when(qp < NQ)
                def _(t=t, b=b):
                    pltpu.make_async_copy(
                        rows[b].at[pl.ds(0, QC)],
                        outv_hbm.at[ibuf.at[t - 2]], gsem[b]).wait()
            if t < NQ // NS + 1:
                q = s + NS * t

                @pl.when(q < NQ)
                def _(q=q, t=t, b=b):
                    pltpu.sync_copy(acc_sh.at[pl.ds(q * QC, QC)],
                                    rows[b].at[pl.ds(0, QC)])
                    for j in range(QC // 16):
                        ibuf[t, pl.ds(j * 16, 16)] = (
                            ibase[pl.ds(j * 16, 16)] + (q * QC * R + r))
                    pltpu.async_copy(rows[b].at[pl.ds(0, QC)],
                                     outv_hbm.at[ibuf.at[t]], gsem[b])
        return carry

    lax.fori_loop(c * 4, 4 + c * 3, rank_pass, 0)

    @pl.when(c == 1)
    def _():
        for t in range(NQ // NS + 3):
            b = t % 2
            if t >= 2:
                qp = s + NS * (t - 2)

                @pl.when(qp < NQ)
                def _(t=t, b=b):
                    pltpu.make_async_copy(
                        rows[b].at[pl.ds(0, QC)],
                        outv_hbm.at[ibuf.at[t - 2]], gsem[b]).wait()
            if t < NQ // NS + 1:
                q = s + NS * t

                @pl.when(q < NQ)
                def _(q=q, t=t, b=b):
                    pltpu.sync_copy(x_hbm.at[pl.ds(q * QC, QC)],
                                    rows[b].at[pl.ds(0, QC)])
                    for j in range(QC // 16):
                        ibuf[t, pl.ds(j * 16, 16)] = (
                            ibase[pl.ds(j * 16, 16)] + (q * QC * R + RM1))
                    pltpu.async_copy(rows[b].at[pl.ds(0, QC)],
                                     outv_hbm.at[ibuf.at[t]], gsem[b])


_hf_call = pl.kernel(
    _hf_body,
    out_type=jax.ShapeDtypeStruct((2, H, D), f32),
    mesh=_mesh,
    scratch_types=[
        pltpu.VMEM((2, CH), i32),
        pltpu.VMEM((2, CH), i32),
        pltpu.VMEM((2, CH), i32),
        pltpu.VMEM((2, CH), i32),
        pltpu.VMEM((CH, D), f32),
        pltpu.VMEM((CH, D), f32),
        pltpu.VMEM((ZC, D), f32),
        [pltpu.SemaphoreType.DMA] * 4,
        [pltpu.SemaphoreType.DMA] * 2,
        [pltpu.SemaphoreType.DMA] * 2,
        pltpu.SemaphoreType.DMA,
        pltpu.VMEM_SHARED((H, D), f32),
    ],
)

_flat_call = pl.kernel(
    _flat_body,
    out_type=jax.ShapeDtypeStruct((H * RM1, D), f32),
    mesh=_mesh,
    scratch_types=[
        pltpu.VMEM((2, 8, D), f32),
        pltpu.VMEM((2, 8, D), f32),
        pltpu.VMEM((8, RM1 * 16), f32),
        pltpu.VMEM((8, RM1 * 16), f32),
        pltpu.VMEM((8 * RM1, D), f32),
        pltpu.VMEM((8 * RM1, D), f32),
        [pltpu.SemaphoreType.DMA] * 2,
        [pltpu.SemaphoreType.DMA] * 2,
    ],
)

_out_call = pl.kernel(
    _out_body,
    out_type=jax.ShapeDtypeStruct((N * R, D), f32),
    mesh=_mesh,
    scratch_types=[
        pltpu.VMEM((2, CH), i32),
        pltpu.VMEM((2, CH), i32),
        pltpu.VMEM((2, CH), i32),
        pltpu.VMEM((2, CH), i32),
        pltpu.VMEM((CH,), i32),
        pltpu.VMEM((CH,), i32),
        pltpu.VMEM((CH, D), f32),
        pltpu.VMEM((CH, D), f32),
        pltpu.VMEM((ZC, D), f32),
        pltpu.VMEM((QC,), i32),
        pltpu.VMEM((NQ // NS + 1, QC), i32),
        [pltpu.SemaphoreType.DMA] * 4,
        [pltpu.SemaphoreType.DMA] * 2,
        [pltpu.SemaphoreType.DMA] * 2,
        pltpu.SemaphoreType.DMA,
        pltpu.VMEM_SHARED((N, D), f32),
    ],
)


@jax.jit
def kernel(x, rank_masks, edge_index, he_idxs):
    rm = rank_masks[:, he_idxs]
    rmb = jnp.broadcast_to(rm[:RM1].T[:, :, None], (H, RM1, 16))
    rmb = rmb.reshape(H, RM1 * 16).astype(f32)
    hfp = _hf_call(x, edge_index)
    flat = _flat_call(hfp, rmb)
    out = _out_call(flat, edge_index, x)
    return out.reshape(N, R, D)
